# baseline (speedup 1.0000x reference)
import jax
import jax.numpy as jnp
from jax.experimental import pallas as pl
from jax.experimental.pallas import tpu as pltpu

_T = 128
_LEVELS = 7
_N = 512
_C = 512
_B = 8


def _bounds(seg_ref):
    a = jnp.clip(jnp.transpose(seg_ref[0]), 0.0, 125.0)
    s0 = jnp.floor(a[0:1, :])
    s1 = jnp.ceil(a[1:2, :])
    s1 = jnp.where(s0 == s1, jnp.ceil(a[1:2, :] + 1.0), s1)
    e0 = jnp.floor(a[2:3, :])
    e1 = jnp.ceil(a[3:4, :])
    e0 = jnp.where(e0 == e1, jnp.floor(a[2:3, :] - 1.0), e0)

    def idx_pair(lo_f, hi_f):
        lo = jnp.maximum(lo_f, 0.0).astype(jnp.int32)
        hi = hi_f.astype(jnp.int32)
        w = hi - lo
        k = ((w >= 2).astype(jnp.int32) + (w >= 4).astype(jnp.int32)
             + (w >= 8).astype(jnp.int32) + (w >= 16).astype(jnp.int32)
             + (w >= 32).astype(jnp.int32) + (w >= 64).astype(jnp.int32))
        two_k = jnp.left_shift(jnp.int32(1), k)
        j1 = k * _T + lo
        j2 = k * _T + hi - two_k
        empty = w < 1
        return j1, j2, empty

    return idx_pair(s0, s1), idx_pair(e0, e1)


def _body(f_ref, seg_ref, out_ref, es_ref, ee_ref):
    b = pl.program_id(0)
    (j1s, j2s, empty_s), (j1e, j2e, empty_e) = _bounds(seg_ref)

    @pl.when(b == 0)
    def _build_onehots():
        iota = jax.lax.broadcasted_iota(jnp.int32, (_LEVELS * _T, 2 * _N), 0)
        js = jnp.concatenate([j1s, j2s], axis=-1)
        je = jnp.concatenate([j1e, j2e], axis=-1)
        es_ref[...] = (iota == js).astype(jnp.bfloat16)
        ee_ref[...] = (iota == je).astype(jnp.bfloat16)

    p = f_ref[0].astype(jnp.bfloat16)
    tables = [p]
    for s in (1, 2, 4, 8, 16, 32):
        shifted = jnp.concatenate([p[:, s:], p[:, :s]], axis=-1)
        p = jnp.maximum(p, shifted)
        tables.append(p)
    table = jnp.concatenate(tables, axis=-1)

    neg_inf = jnp.float32(-jnp.inf)

    def half(tab_half, em, empty):
        g = jnp.dot(tab_half, em, preferred_element_type=jnp.float32)
        out = jnp.maximum(g[:, :_N], g[:, _N:])
        return jnp.where(empty, neg_inf, out)

    out_ref[0, : _C // 2, :] = half(table[: _C // 2], es_ref[...], empty_s)
    out_ref[0, _C // 2 :, :] = half(table[_C // 2 :], ee_ref[...], empty_e)


@jax.jit
def _run(feature, seg_t):
    onehot_scratch = pltpu.VMEM((_LEVELS * _T, 2 * _N), jnp.bfloat16)
    return pl.pallas_call(
        _body,
        grid=(_B,),
        in_specs=[
            pl.BlockSpec((1, _C, _T), lambda b: (b, 0, 0)),
            pl.BlockSpec((1, _N, 4), lambda b: (0, 0, 0)),
        ],
        out_specs=pl.BlockSpec((1, _C, _N), lambda b: (b, 0, 0)),
        out_shape=jax.ShapeDtypeStruct((_B, _C, _N), jnp.float32),
        scratch_shapes=[onehot_scratch] * 2,
    )(feature, seg_t)


def kernel(feature, segments):
    return _run(feature, segments)


# speedup vs baseline: 1.0441x; 1.0441x over previous
import jax
import jax.numpy as jnp
from jax.experimental import pallas as pl
from jax.experimental.pallas import tpu as pltpu

_T = 128
_LEVELS = 7
_N = 512
_C = 512
_B = 8


def _bounds(seg_ref):
    a = jnp.clip(jnp.transpose(seg_ref[0]), 0.0, 125.0)
    s0 = jnp.floor(a[0:1, :])
    s1 = jnp.ceil(a[1:2, :])
    s1 = jnp.where(s0 == s1, jnp.ceil(a[1:2, :] + 1.0), s1)
    e0 = jnp.floor(a[2:3, :])
    e1 = jnp.ceil(a[3:4, :])
    e0 = jnp.where(e0 == e1, jnp.floor(a[2:3, :] - 1.0), e0)

    def idx_pair(lo_f, hi_f):
        lo = jnp.maximum(lo_f, 0.0).astype(jnp.int32)
        hi = hi_f.astype(jnp.int32)
        w = hi - lo
        k = ((w >= 2).astype(jnp.int32) + (w >= 4).astype(jnp.int32)
             + (w >= 8).astype(jnp.int32) + (w >= 16).astype(jnp.int32)
             + (w >= 32).astype(jnp.int32) + (w >= 64).astype(jnp.int32))
        two_k = jnp.left_shift(jnp.int32(1), k)
        j1 = k * _T + lo
        j2 = k * _T + hi - two_k
        empty = w < 1
        return j1, j2, empty

    return idx_pair(s0, s1), idx_pair(e0, e1)


def _body(f_ref, seg_ref, out_ref):
    (j1s, j2s, empty_s), (j1e, j2e, empty_e) = _bounds(seg_ref)

    iota = jax.lax.broadcasted_iota(jnp.int32, (_LEVELS * _T, 2 * _N), 0)
    es = (iota == jnp.concatenate([j1s, j2s], -1)).astype(jnp.bfloat16)
    ee = (iota == jnp.concatenate([j1e, j2e], -1)).astype(jnp.bfloat16)

    p = f_ref[0]
    tables = [p]
    for s in (1, 2, 4, 8, 16, 32):
        shifted = jnp.concatenate([p[:, s:], p[:, :s]], axis=-1)
        p = jnp.maximum(p, shifted)
        tables.append(p)
    table = jnp.concatenate(tables, axis=-1).astype(jnp.bfloat16)

    neg_inf = jnp.float32(-jnp.inf)

    def half(tab_half, em, empty):
        g = jnp.dot(tab_half, em, preferred_element_type=jnp.float32)
        out = jnp.maximum(g[:, :_N], g[:, _N:])
        return jnp.where(empty, neg_inf, out)

    out_ref[0, : _C // 2, :] = half(table[: _C // 2], es, empty_s)
    out_ref[0, _C // 2 :, :] = half(table[_C // 2 :], ee, empty_e)


@jax.jit
def _run(feature, segments):
    return pl.pallas_call(
        _body,
        grid=(_B,),
        in_specs=[
            pl.BlockSpec((1, _C, _T), lambda b: (b, 0, 0)),
            pl.BlockSpec((1, _N, 4), lambda b: (0, 0, 0)),
        ],
        out_specs=pl.BlockSpec((1, _C, _N), lambda b: (b, 0, 0)),
        out_shape=jax.ShapeDtypeStruct((_B, _C, _N), jnp.float32),
    )(feature, segments)


def kernel(feature, segments):
    return _run(feature, segments)


# speedup vs baseline: 1.2612x; 1.2079x over previous
import jax
import jax.numpy as jnp
from jax.experimental import pallas as pl
from jax.experimental.pallas import tpu as pltpu

_T = 128
_LEVELS = 7
_N = 512
_C = 512
_B = 8
_BB = 2


def _bounds(seg_ref):
    a = jnp.clip(seg_ref[...], 0.0, 125.0)
    s0 = jnp.floor(a[0:1, :])
    s1 = jnp.ceil(a[1:2, :])
    s1 = jnp.where(s0 == s1, jnp.ceil(a[1:2, :] + 1.0), s1)
    e0 = jnp.floor(a[2:3, :])
    e1 = jnp.ceil(a[3:4, :])
    e0 = jnp.where(e0 == e1, jnp.floor(a[2:3, :] - 1.0), e0)

    def idx_pair(lo_f, hi_f):
        lo = jnp.maximum(lo_f, 0.0).astype(jnp.int32)
        hi = hi_f.astype(jnp.int32)
        w = hi - lo
        k = ((w >= 2).astype(jnp.int32) + (w >= 4).astype(jnp.int32)
             + (w >= 8).astype(jnp.int32) + (w >= 16).astype(jnp.int32)
             + (w >= 32).astype(jnp.int32) + (w >= 64).astype(jnp.int32))
        two_k = jnp.left_shift(jnp.int32(1), k)
        j1 = k * _T + lo
        j2 = k * _T + hi - two_k
        empty = w < 1
        return j1, j2, empty

    return idx_pair(s0, s1), idx_pair(e0, e1)


def _body(f_ref, seg_ref, out_ref):
    (j1s, j2s, empty_s), (j1e, j2e, empty_e) = _bounds(seg_ref)

    iota = jax.lax.broadcasted_iota(jnp.int32, (_LEVELS * _T, 2 * _N), 0)
    es = (iota == jnp.concatenate([j1s, j2s], -1)).astype(jnp.bfloat16)
    ee = (iota == jnp.concatenate([j1e, j2e], -1)).astype(jnp.bfloat16)

    neg_inf = jnp.float32(-jnp.inf)

    for bb in range(_BB):
        p = f_ref[bb]
        tables = [p]
        for s in (1, 2, 4, 8, 16, 32):
            shifted = jnp.concatenate([p[:, s:], p[:, :s]], axis=-1)
            p = jnp.maximum(p, shifted)
            tables.append(p)
        table = jnp.concatenate(tables, -1).astype(jnp.bfloat16)

        def half(tab_half, em, empty):
            g = jnp.dot(tab_half, em, preferred_element_type=jnp.float32)
            out = jnp.maximum(g[:, :_N], g[:, _N:])
            return jnp.where(empty, neg_inf, out)

        out_ref[bb, : _C // 2, :] = half(table[: _C // 2], es, empty_s)
        out_ref[bb, _C // 2 :, :] = half(table[_C // 2 :], ee, empty_e)


@jax.jit
def _run(feature, seg_t):
    return pl.pallas_call(
        _body,
        grid=(_B // _BB,),
        in_specs=[
            pl.BlockSpec((_BB, _C, _T), lambda b: (b, 0, 0)),
            pl.BlockSpec((4, _N), lambda b: (0, 0)),
        ],
        out_specs=pl.BlockSpec((_BB, _C, _N), lambda b: (b, 0, 0)),
        out_shape=jax.ShapeDtypeStruct((_B, _C, _N), jnp.float32),
    )(feature, seg_t)


def kernel(feature, segments):
    seg_t = segments[0].T
    return _run(feature, seg_t)


# speedup vs baseline: 1.2734x; 1.0096x over previous
import jax
import jax.numpy as jnp
from jax.experimental import pallas as pl
from jax.experimental.pallas import tpu as pltpu

_T = 128
_LEVELS = 7
_N = 512
_C = 512
_B = 8
_BB = 4


def _bounds(seg_ref):
    a = jnp.clip(seg_ref[...], 0.0, 125.0)
    s0 = jnp.floor(a[0:1, :])
    s1 = jnp.ceil(a[1:2, :])
    s1 = jnp.where(s0 == s1, jnp.ceil(a[1:2, :] + 1.0), s1)
    e0 = jnp.floor(a[2:3, :])
    e1 = jnp.ceil(a[3:4, :])
    e0 = jnp.where(e0 == e1, jnp.floor(a[2:3, :] - 1.0), e0)

    def idx_pair(lo_f, hi_f):
        lo = jnp.maximum(lo_f, 0.0).astype(jnp.int32)
        hi = hi_f.astype(jnp.int32)
        w = hi - lo
        k = ((w >= 2).astype(jnp.int32) + (w >= 4).astype(jnp.int32)
             + (w >= 8).astype(jnp.int32) + (w >= 16).astype(jnp.int32)
             + (w >= 32).astype(jnp.int32) + (w >= 64).astype(jnp.int32))
        two_k = jnp.left_shift(jnp.int32(1), k)
        j1 = k * _T + lo
        j2 = k * _T + hi - two_k
        empty = w < 1
        return j1, j2, empty

    return idx_pair(s0, s1), idx_pair(e0, e1)


def _body(f_ref, seg_ref, out_ref):
    (j1s, j2s, empty_s), (j1e, j2e, empty_e) = _bounds(seg_ref)

    iota = jax.lax.broadcasted_iota(jnp.int32, (_LEVELS * _T, 2 * _N), 0)
    es = (iota == jnp.concatenate([j1s, j2s], -1)).astype(jnp.bfloat16)
    ee = (iota == jnp.concatenate([j1e, j2e], -1)).astype(jnp.bfloat16)

    neg_inf = jnp.float32(-jnp.inf)

    for bb in range(_BB):
        p = f_ref[bb]
        tables = [p]
        for s in (1, 2, 4, 8, 16, 32):
            shifted = jnp.concatenate([p[:, s:], p[:, :s]], axis=-1)
            p = jnp.maximum(p, shifted)
            tables.append(p)
        table = jnp.concatenate(tables, -1).astype(jnp.bfloat16)

        def half(tab_half, em, empty):
            g = jnp.dot(tab_half, em, preferred_element_type=jnp.float32)
            out = jnp.maximum(g[:, :_N], g[:, _N:])
            return jnp.where(empty, neg_inf, out)

        out_ref[bb, : _C // 2, :] = half(table[: _C // 2], es, empty_s)
        out_ref[bb, _C // 2 :, :] = half(table[_C // 2 :], ee, empty_e)


@jax.jit
def _run(feature, seg_t):
    return pl.pallas_call(
        _body,
        grid=(_B // _BB,),
        in_specs=[
            pl.BlockSpec((_BB, _C, _T), lambda b: (b, 0, 0)),
            pl.BlockSpec((4, _N), lambda b: (0, 0)),
        ],
        out_specs=pl.BlockSpec((_BB, _C, _N), lambda b: (b, 0, 0)),
        out_shape=jax.ShapeDtypeStruct((_B, _C, _N), jnp.float32),
    )(feature, seg_t)


def kernel(feature, segments):
    seg_t = segments[0].T
    return _run(feature, seg_t)
